# Initial kernel scaffold; baseline (speedup 1.0000x reference)
import jax
import jax.numpy as jnp
from jax.experimental import pallas as pl


def kernel(keys, table):
    raise NotImplementedError("write your pallas kernel here")


# speedup vs baseline: 5.6618x; 5.6618x over previous
import functools

import jax
import jax.numpy as jnp
from jax import lax
from jax.experimental import pallas as pl
from jax.experimental.pallas import tpu as pltpu
from jax.experimental.pallas import tpu_sc as plsc

_NUM_EMB = 1024
_EMB_SIZE = 8
_B = 16384 * 26
_TABLE_W = _NUM_EMB * _EMB_SIZE

_info = plsc.get_sparse_core_info()
_NC, _NS, _L = _info.num_cores, _info.num_subcores, _info.num_lanes
_NW = _NC * _NS
_B_PER_W = _B // _NW
_N_CHUNKS = 8
_GROUPS_PER_CHUNK = _B_PER_W // (_N_CHUNKS * _L)
_CHUNK_W = _B_PER_W * _EMB_SIZE // _N_CHUNKS


@functools.partial(
    pl.kernel,
    out_type=jax.ShapeDtypeStruct((_B * _EMB_SIZE,), jnp.float32),
    mesh=plsc.VectorSubcoreMesh(core_axis_name="c", subcore_axis_name="s"),
    compiler_params=pltpu.CompilerParams(needs_layout_passes=False),
    scratch_types=[
        pltpu.VMEM((_TABLE_W,), jnp.float32),
        pltpu.VMEM((_B_PER_W,), jnp.int32),
        pltpu.VMEM((_B_PER_W * _EMB_SIZE,), jnp.float32),
        pltpu.SemaphoreType.DMA,
        pltpu.SemaphoreType.DMA,
    ],
)
def _gather_kernel(keys_hbm, table_hbm, out_hbm, table_v, idx_v, out_v,
                   in_sem, out_sem):
    wid = lax.axis_index("s") * _NC + lax.axis_index("c")
    base = wid * _B_PER_W

    tbl_cp = pltpu.async_copy(table_hbm, table_v, in_sem)
    key_cp = pltpu.async_copy(keys_hbm.at[pl.ds(base, _B_PER_W)], idx_v,
                              in_sem)
    tbl_cp.wait()
    key_cp.wait()

    iota8 = lax.iota(jnp.int32, _L) * _EMB_SIZE
    out_cps = []
    for chunk in range(_N_CHUNKS):
        g0 = chunk * _GROUPS_PER_CHUNK

        @plsc.parallel_loop(0, _GROUPS_PER_CHUNK)
        def _body(g, _g0=g0):
            keys16 = idx_v[pl.ds((_g0 + g) * _L, _L)]
            keybase = keys16 * _EMB_SIZE
            posbase = iota8 + (_g0 + g) * (_L * _EMB_SIZE)
            for c in range(_EMB_SIZE):
                col = plsc.load_gather(table_v, [keybase + c])
                plsc.store_scatter(out_v, [posbase + c], col)

        out_cps.append(
            pltpu.async_copy(
                out_v.at[pl.ds(chunk * _CHUNK_W, _CHUNK_W)],
                out_hbm.at[pl.ds(base * _EMB_SIZE + chunk * _CHUNK_W,
                                 _CHUNK_W)],
                out_sem,
            ))
    for cp in out_cps:
        cp.wait()


def kernel(keys, table):
    flat_keys = keys.reshape(_B)
    flat_table = table.reshape(_TABLE_W)
    out = _gather_kernel(flat_keys, flat_table)
    return out.reshape(keys.shape[0], keys.shape[1], _EMB_SIZE)


# speedup vs baseline: 45.3712x; 8.0135x over previous
import functools

import jax
import jax.numpy as jnp
from jax import lax
from jax.experimental import pallas as pl
from jax.experimental.pallas import tpu as pltpu
from jax.experimental.pallas import tpu_sc as plsc

_NUM_EMB = 1024
_EMB_SIZE = 8
_NB = 16384
_NSLOT = 26
_TABLE_W = _NUM_EMB * _EMB_SIZE

_info = plsc.get_sparse_core_info()
_NC, _NS, _L = _info.num_cores, _info.num_subcores, _info.num_lanes
_NW = _NC * _NS
_BW = _NB // _NW
_GROUPS = _BW // _L


@functools.partial(
    pl.kernel,
    out_type=jax.ShapeDtypeStruct((_NSLOT, _EMB_SIZE, _NB), jnp.float32),
    mesh=plsc.VectorSubcoreMesh(core_axis_name="c", subcore_axis_name="s"),
    compiler_params=pltpu.CompilerParams(needs_layout_passes=False),
    scratch_types=[
        pltpu.VMEM((_TABLE_W,), jnp.float32),
        pltpu.VMEM((_NSLOT, _BW), jnp.int32),
        pltpu.VMEM((_NSLOT, _EMB_SIZE, _BW), jnp.float32),
        pltpu.SemaphoreType.DMA,
        pltpu.SemaphoreType.DMA,
    ],
)
def _gather_kernel(keys_hbm, table_hbm, out_hbm, table_v, keys_v, out_v,
                   in_sem, out_sem):
    wid = lax.axis_index("s") * _NC + lax.axis_index("c")
    b0 = wid * _BW

    tbl_cp = pltpu.async_copy(table_hbm, table_v, in_sem)
    key_cp = pltpu.async_copy(keys_hbm.at[:, pl.ds(b0, _BW)], keys_v, in_sem)
    tbl_cp.wait()
    key_cp.wait()

    iota = lax.iota(jnp.int32, _L)
    cvecs = [jnp.full((_L,), c, jnp.int32) for c in range(_EMB_SIZE)]
    out_cps = []
    for s in range(_NSLOT):
        sfull = jnp.full((_L,), s, jnp.int32)

        @plsc.parallel_loop(0, _GROUPS)
        def _body(g, _sfull=sfull):
            bvec = g * _L + iota
            keys16 = plsc.load_gather(keys_v, [_sfull, bvec])
            for c in range(_EMB_SIZE):
                col = plsc.load_gather(table_v, [keys16 + c * _NUM_EMB])
                plsc.store_scatter(out_v, [_sfull, cvecs[c], bvec], col)

        out_cps.append(
            pltpu.async_copy(out_v.at[s], out_hbm.at[s, :, pl.ds(b0, _BW)],
                             out_sem))
    for cp in out_cps:
        cp.wait()


def kernel(keys, table):
    keys_t = keys.T
    table_cm = table.T.reshape(_TABLE_W)
    p = _gather_kernel(keys_t, table_cm)
    return p.transpose(2, 0, 1)
